# baseline (speedup 1.0000x reference)
import functools

import jax
import jax.numpy as jnp
from jax import lax
from jax.experimental import pallas as pl
from jax.experimental.pallas import tpu as pltpu
from jax.experimental.pallas import tpu_sc as plsc

_LN2 = 0.6931471805599453
_LOGC = (3.193085771957538e-05, 1.441267074216371, -0.7057026209300269,
         0.4087189439210336, -0.18772049275771308, 0.0434283633315784)

_SC_FRAC = 0.25


def _log2_poly(w):
    bits = plsc.bitcast(w, jnp.int32)
    e = lax.shift_right_logical(bits, 23) - 127
    m_bits = jnp.bitwise_or(jnp.bitwise_and(bits, 0x007FFFFF), 0x3F800000)
    t = plsc.bitcast(m_bits, jnp.float32) - 1.0
    p = jnp.full((16,), _LOGC[5], jnp.float32)
    for c in (_LOGC[4], _LOGC[3], _LOGC[2], _LOGC[1], _LOGC[0]):
        p = p * t + c
    return e.astype(jnp.float32) + p


def _sc_loss_body(start, rows_per, C, x_hbm, t_hbm, out_hbm,
                  xrows, trows, accv):
    nc = plsc.get_sparse_core_info().num_cores
    wid = lax.axis_index("s") * nc + lax.axis_index("c")
    base = start + wid * rows_per
    lane = lax.iota(jnp.int32, 16)

    pltpu.sync_copy(x_hbm.at[pl.ds(base, rows_per)], xrows)
    pltpu.sync_copy(t_hbm.at[pl.ds(base, rows_per)], trows)

    def group(g, carry):
        corr_acc, b_acc, l_acc = carry
        rowidx = g * 16 + lane
        a = jnp.zeros((16,), jnp.float32)
        p = jnp.zeros((16,), jnp.float32)
        no = jnp.zeros((16,), jnp.int32)
        bsum = b_acc
        lsum = l_acc
        for c in range(C):
            cc = jnp.full((16,), c, jnp.int32)
            x = plsc.load_gather(xrows, [rowidx, cc])
            t = plsc.load_gather(trows, [rowidx, cc])
            absent = t == 0
            u = jnp.exp(-x)
            w = 1.0 + u
            s = 1.0 / w
            es = jnp.exp(s)
            a = a + jnp.where(absent, es, 0.0)
            p = p + jnp.where(absent, 0.0, 1.0 / es)
            no = no + t
            bsum = bsum + jnp.where(absent, x, 0.0)
            lsum = lsum + _log2_poly(w)
        nof = no.astype(jnp.float32)
        den = nof * (float(C) - nof)
        per = jnp.where(den > 0.0, (a * p) / jnp.maximum(den, 1.0), 0.0)
        return corr_acc + per, bsum, lsum

    zero = jnp.zeros((16,), jnp.float32)
    corr_acc, b_acc, l_acc = lax.fori_loop(
        0, rows_per // 16, group, (zero, zero, zero))
    accv[pl.ds(0, 16)] = corr_acc
    accv[pl.ds(16, 16)] = b_acc
    accv[pl.ds(32, 16)] = l_acc
    pltpu.sync_copy(accv, out_hbm.at[wid])


def _sc_loss_partials(logits, targets, start, rows):
    B, C = logits.shape
    info = plsc.get_sparse_core_info()
    nw = info.num_cores * info.num_subcores
    rows_per = rows // nw
    mesh = plsc.VectorSubcoreMesh(core_axis_name="c", subcore_axis_name="s")
    k = functools.partial(
        pl.kernel,
        mesh=mesh,
        compiler_params=pltpu.CompilerParams(
            needs_layout_passes=False, use_tc_tiling_on_sc=True),
        out_type=jax.ShapeDtypeStruct((nw, 48), jnp.float32),
        scratch_types=[
            pltpu.VMEM((rows_per, C), jnp.float32),
            pltpu.VMEM((rows_per, C), jnp.int32),
            pltpu.VMEM((48,), jnp.float32),
        ],
    )(functools.partial(_sc_loss_body, start, rows_per, C))
    return k(logits, targets)


def _tc_loss_body(x_ref, t_ref, o_ref):
    x = x_ref[:]
    y = t_ref[:].astype(jnp.float32)
    C = x.shape[1]

    u = jnp.exp(-x)
    w = 1.0 + u
    bce = jnp.sum(x * (1.0 - y) + jnp.log(w))
    s = 1.0 / w
    es = jnp.exp(s)
    a = jnp.sum(jnp.where(y == 0.0, es, 0.0), axis=1)
    p = jnp.sum(jnp.where(y == 0.0, 0.0, 1.0 / es), axis=1)
    n_o = jnp.sum(y, axis=1)
    den = n_o * (C - n_o)
    per = jnp.where(den > 0.0, (a * p) / jnp.maximum(den, 1.0), 0.0)
    corr = jnp.sum(per)

    i = pl.program_id(0)

    @pl.when(i == 0)
    def _():
        o_ref[:] = jnp.zeros_like(o_ref)

    col = lax.broadcasted_iota(jnp.int32, (1, 128), 1)
    o_ref[:] += jnp.where(col == 0, bce, 0.0) + jnp.where(col == 1, corr, 0.0)


def _tc_loss_partials(logits, targets, rows, grid=4):
    B, C = logits.shape
    blk = rows // grid
    out = pl.pallas_call(
        _tc_loss_body,
        grid=(grid,),
        in_specs=[
            pl.BlockSpec((blk, C), lambda i: (i, 0)),
            pl.BlockSpec((blk, C), lambda i: (i, 0)),
        ],
        out_specs=pl.BlockSpec((1, 128), lambda i: (0, 0)),
        out_shape=jax.ShapeDtypeStruct((1, 128), jnp.float32),
    )(logits, targets)
    return out[0, 0], out[0, 1]


def kernel(logits, targets):
    B, C = logits.shape
    sc_rows = int(B * _SC_FRAC) // 512 * 512
    tc_rows = B - sc_rows
    sc_parts = _sc_loss_partials(logits, targets, tc_rows, sc_rows)
    bce_tc, corr_tc = _tc_loss_partials(logits, targets, tc_rows)
    corr_sum = corr_tc + jnp.sum(sc_parts[:, 0:16])
    bce_sum = (bce_tc + jnp.sum(sc_parts[:, 16:32])
               + _LN2 * jnp.sum(sc_parts[:, 32:48]))
    bce_mean = bce_sum / (B * C)
    corr_mean = corr_sum / B
    return 0.8 * bce_mean + 0.2 * corr_mean


# speedup vs baseline: 1.5978x; 1.5978x over previous
import jax
import jax.numpy as jnp
from jax import lax
from jax.experimental import pallas as pl


def _loss_body(x_ref, t_ref, o_ref):
    x = x_ref[:]
    y = t_ref[:].astype(jnp.float32)
    C = x.shape[1]

    u = jnp.exp(-x)
    w = 1.0 + u
    bce = jnp.sum(x * (1.0 - y) + jnp.log(w))
    s = 1.0 / w
    es = jnp.exp(s)
    a = jnp.sum(jnp.where(y == 0.0, es, 0.0), axis=1)
    p = jnp.sum(jnp.where(y == 0.0, 0.0, 1.0 / es), axis=1)
    n_o = jnp.sum(y, axis=1)
    n_z = C - n_o
    den = n_o * n_z
    per = jnp.where(den > 0.0, (a * p) / jnp.maximum(den, 1.0), 0.0)
    corr = jnp.sum(per)

    i = pl.program_id(0)

    @pl.when(i == 0)
    def _():
        o_ref[:] = jnp.zeros_like(o_ref)

    col = lax.broadcasted_iota(jnp.int32, (1, 128), 1)
    o_ref[:] += jnp.where(col == 0, bce, 0.0) + jnp.where(col == 1, corr, 0.0)


def kernel(logits, targets, grid=4):
    B, C = logits.shape
    blk = B // grid
    out = pl.pallas_call(
        _loss_body,
        grid=(grid,),
        in_specs=[
            pl.BlockSpec((blk, C), lambda i: (i, 0)),
            pl.BlockSpec((blk, C), lambda i: (i, 0)),
        ],
        out_specs=pl.BlockSpec((1, 128), lambda i: (0, 0)),
        out_shape=jax.ShapeDtypeStruct((1, 128), jnp.float32),
    )(logits, targets)
    bce_mean = out[0, 0] / (B * C)
    corr_mean = out[0, 1] / B
    return 0.8 * bce_mean + 0.2 * corr_mean
